# Initial kernel scaffold; baseline (speedup 1.0000x reference)
import functools

import jax
import jax.numpy as jnp
from jax import lax
from jax.experimental import pallas as pl
from jax.experimental.pallas import tpu as pltpu
from jax.experimental.pallas import tpu_sc as plsc

N = 10000
E = 320000
D = 128
H = 4
HD = 32

NC = 2
NS = 16
NW = NC * NS

CH = 128
NCHUNK = E // CH
ITERS = -(-NCHUNK // NW)

ACCW = 144

_sc_mesh = plsc.VectorSubcoreMesh(core_axis_name="c", subcore_axis_name="s")


def _proj_body(nodes_ref, wi_ref, wj_ref, bi_ref, bj_ref, s_ref, r_ref):
    x = nodes_ref[...]
    s_ref[...] = (
        jnp.dot(x, wi_ref[...], preferred_element_type=jnp.float32) + bi_ref[...]
    )
    r_ref[...] = (
        jnp.dot(x, wj_ref[...], preferred_element_type=jnp.float32) + bj_ref[...]
    )


def _proj(nodes, wi2, wj2, bi2, bj2):
    bn = 1250
    return pl.pallas_call(
        _proj_body,
        grid=(N // bn,),
        in_specs=[
            pl.BlockSpec((bn, D), lambda i: (i, 0)),
            pl.BlockSpec((D, D), lambda i: (0, 0)),
            pl.BlockSpec((D, D), lambda i: (0, 0)),
            pl.BlockSpec((1, D), lambda i: (0, 0)),
            pl.BlockSpec((1, D), lambda i: (0, 0)),
        ],
        out_specs=[
            pl.BlockSpec((bn, D), lambda i: (i, 0)),
            pl.BlockSpec((bn, D), lambda i: (i, 0)),
        ],
        out_shape=[
            jax.ShapeDtypeStruct((N, D), jnp.float32),
            jax.ShapeDtypeStruct((N, D), jnp.float32),
        ],
    )(nodes, wi2, wj2, bi2, bj2)


@functools.partial(
    pl.kernel,
    mesh=_sc_mesh,
    out_type=jax.ShapeDtypeStruct((E, D), jnp.float32),
    scratch_types=[
        pltpu.VMEM((CH,), jnp.int32),
        pltpu.VMEM((CH,), jnp.int32),
        pltpu.VMEM((CH, D), jnp.float32),
        pltpu.VMEM((CH, D), jnp.float32),
        pltpu.SemaphoreType.DMA,
    ],
)
def _edges_x(s_hbm, r_hbm, send_hbm, recv_hbm, x_hbm, idx_s, idx_r, s_buf, r_buf, sem):
    wid = lax.axis_index("s") * NC + lax.axis_index("c")

    @pl.loop(0, ITERS)
    def _(i):
        c = wid + i * NW

        @pl.when(c < NCHUNK)
        def _():
            base = c * CH
            pltpu.sync_copy(send_hbm.at[pl.ds(base, CH)], idx_s)
            pltpu.sync_copy(recv_hbm.at[pl.ds(base, CH)], idx_r)
            cp1 = pltpu.async_copy(s_hbm.at[idx_s], s_buf, sem)
            cp2 = pltpu.async_copy(r_hbm.at[idx_r], r_buf, sem)
            cp1.wait()
            cp2.wait()

            @pl.loop(0, CH)
            def _(e):
                for g in range(D // 16):
                    slc = (pl.ds(e, 1), pl.ds(g * 16, 16))
                    s_buf.at[*slc][...] = s_buf.at[*slc][...] + r_buf.at[*slc][...]

            pltpu.sync_copy(s_buf, x_hbm.at[pl.ds(base, CH)])


def _logits_body(x_ref, m_ref, z_ref):
    x = x_ref[...]
    m = x * jnp.tanh(jax.nn.softplus(x))
    l16 = jnp.dot(m, m_ref[...], preferred_element_type=jnp.float32)
    mask = (lax.broadcasted_iota(jnp.int32, l16.shape, 1) < H).astype(jnp.float32)
    z_ref[...] = jnp.exp(l16) * mask


def _logits(x, m16):
    be = 2560
    return pl.pallas_call(
        _logits_body,
        grid=(E // be,),
        in_specs=[
            pl.BlockSpec((be, D), lambda i: (i, 0)),
            pl.BlockSpec((D, 16), lambda i: (0, 0)),
        ],
        out_specs=pl.BlockSpec((be, 16), lambda i: (i, 0)),
        out_shape=jax.ShapeDtypeStruct((E, 16), jnp.float32),
    )(x, m16)


@functools.partial(
    pl.kernel,
    mesh=_sc_mesh,
    out_type=jax.ShapeDtypeStruct((NC, N, ACCW), jnp.float32),
    scratch_types=[
        pltpu.VMEM((CH,), jnp.int32),
        pltpu.VMEM((CH,), jnp.int32),
        pltpu.VMEM((CH, D), jnp.float32),
        pltpu.VMEM((CH, 16), jnp.float32),
        pltpu.VMEM((CH, ACCW), jnp.float32),
        pltpu.VMEM_SHARED((N, ACCW), jnp.float32),
        pltpu.SemaphoreType.DMA,
    ],
)
def _aggregate(
    s_hbm, send_hbm, recv_hbm, z_hbm, part_hbm,
    idx_s, idx_r, s_buf, z_buf, msg_buf, acc, sem,
):
    cid = lax.axis_index("c")
    sid = lax.axis_index("s")
    wid = sid * NC + cid

    @pl.when(sid == 0)
    def _():
        @pl.loop(0, CH)
        def _(e):
            for g in range(ACCW // 16):
                msg_buf.at[pl.ds(e, 1), pl.ds(g * 16, 16)][...] = jnp.zeros(
                    (1, 16), jnp.float32
                )

        @pl.loop(0, N // CH)
        def _(j):
            pltpu.sync_copy(msg_buf, acc.at[pl.ds(j * CH, CH)])

        rem = N - (N // CH) * CH
        if rem:
            pltpu.sync_copy(
                msg_buf.at[pl.ds(0, rem)], acc.at[pl.ds((N // CH) * CH, rem)]
            )

    plsc.subcore_barrier()

    @pl.loop(0, ITERS)
    def _(i):
        c = wid + i * NW

        @pl.when(c < NCHUNK)
        def _():
            base = c * CH
            pltpu.sync_copy(send_hbm.at[pl.ds(base, CH)], idx_s)
            pltpu.sync_copy(recv_hbm.at[pl.ds(base, CH)], idx_r)
            pltpu.sync_copy(z_hbm.at[pl.ds(base, CH)], z_buf)
            pltpu.async_copy(s_hbm.at[idx_s], s_buf, sem).wait()

            @pl.loop(0, CH)
            def _(e):
                zr = z_buf.at[pl.ds(e, 1), pl.ds(0, 16)][...]
                msg_buf.at[pl.ds(e, 1), pl.ds(D, 16)][...] = zr
                for h in range(H):
                    zv = jnp.broadcast_to(z_buf[e, h], (1, 16))
                    for g in range(2):
                        slc = (pl.ds(e, 1), pl.ds(h * HD + g * 16, 16))
                        msg_buf.at[*slc][...] = s_buf.at[*slc][...] * zv

            pltpu.sync_copy(msg_buf, acc.at[idx_r], add=True)

    plsc.subcore_barrier()

    @pl.when(sid < NS - 1)
    def _():
        pltpu.sync_copy(
            acc.at[pl.ds(sid * 624, 624)], part_hbm.at[cid, pl.ds(sid * 624, 624)]
        )

    @pl.when(sid == NS - 1)
    def _():
        pltpu.sync_copy(
            acc.at[pl.ds((NS - 1) * 624, N - (NS - 1) * 624)],
            part_hbm.at[cid, pl.ds((NS - 1) * 624, N - (NS - 1) * 624)],
        )


def _final_body(p_ref, o_ref):
    p = p_ref[...]
    t = p[0] + p[1]
    num = t[:, :D]
    den4 = t[:, D : D + H]
    row = lax.broadcasted_iota(jnp.int32, (H, D), 0)
    col = lax.broadcasted_iota(jnp.int32, (H, D), 1)
    pat = (col // HD == row).astype(jnp.float32)
    dent = jnp.dot(den4, pat, preferred_element_type=jnp.float32)
    o_ref[...] = num / jnp.maximum(dent, 1e-37)


def _finalize(parts):
    bn = 1250
    return pl.pallas_call(
        _final_body,
        grid=(N // bn,),
        in_specs=[pl.BlockSpec((NC, bn, ACCW), lambda i: (0, i, 0))],
        out_specs=pl.BlockSpec((bn, D), lambda i: (i, 0)),
        out_shape=jax.ShapeDtypeStruct((N, D), jnp.float32),
    )(parts)


def kernel(nodes, senders, receivers, Wi, bi, Wj, bj, Wa, ba):
    wi2 = Wi.reshape(D, H * HD)
    wj2 = Wj.reshape(D, H * HD)
    bi2 = bi.reshape(1, H * HD)
    bj2 = bj.reshape(1, H * HD)
    eye = jnp.eye(H, dtype=jnp.float32)
    m4 = (eye[:, None, :] * Wa[:, 0][None, :, None]).reshape(H * HD, H)
    m16 = jnp.pad(m4, ((0, 0), (0, 16 - H)))

    s, r = _proj(nodes, wi2, wj2, bi2, bj2)
    x = _edges_x(s, r, senders, receivers)
    z = _logits(x, m16)
    parts = _aggregate(s, senders, receivers, z)
    return _finalize(parts)


# speedup vs baseline: 28.8860x; 28.8860x over previous
import functools

import jax
import jax.numpy as jnp
from jax import lax
from jax.experimental import pallas as pl
from jax.experimental.pallas import tpu as pltpu
from jax.experimental.pallas import tpu_sc as plsc

N = 10000
E = 320000
D = 128
H = 4
HD = 32

NC = 2
NS = 16
NW = NC * NS

CH = 128
NCHUNK = E // CH
ITERS = -(-NCHUNK // NW)

_sc_mesh = plsc.VectorSubcoreMesh(core_axis_name="c", subcore_axis_name="s")


def _zero_buf(buf, rows, cols):
    @pl.loop(0, rows)
    def _(e):
        for g in range(cols // 16):
            buf.at[pl.ds(e, 1), pl.ds(g * 16, 16)][...] = jnp.zeros(
                (1, 16), jnp.float32
            )


def _zero_acc(sid, msg_buf, acc):
    @pl.when(sid == 0)
    def _():
        @pl.loop(0, N // CH)
        def _(j):
            pltpu.sync_copy(msg_buf, acc.at[pl.ds(j * CH, CH)])

        rem = N - (N // CH) * CH
        if rem:
            pltpu.sync_copy(
                msg_buf.at[pl.ds(0, rem)], acc.at[pl.ds((N // CH) * CH, rem)]
            )


def _dump_acc(cid, sid, acc, part_hbm):
    per = 624

    @pl.when(sid < NS - 1)
    def _():
        pltpu.sync_copy(
            acc.at[pl.ds(sid * per, per)], part_hbm.at[cid, pl.ds(sid * per, per)]
        )

    @pl.when(sid == NS - 1)
    def _():
        pltpu.sync_copy(
            acc.at[pl.ds((NS - 1) * per, N - (NS - 1) * per)],
            part_hbm.at[cid, pl.ds((NS - 1) * per, N - (NS - 1) * per)],
        )


def _proj_body(nodes_ref, wi_ref, wj_ref, bi_ref, bj_ref, s_ref, r_ref):
    x = nodes_ref[...]
    s_ref[...] = (
        jnp.dot(x, wi_ref[...], preferred_element_type=jnp.float32) + bi_ref[...]
    )
    r_ref[...] = (
        jnp.dot(x, wj_ref[...], preferred_element_type=jnp.float32) + bj_ref[...]
    )


def _proj(nodes, wi2, wj2, bi2, bj2):
    bn = 2000
    return pl.pallas_call(
        _proj_body,
        grid=(N // bn,),
        in_specs=[
            pl.BlockSpec((bn, D), lambda i: (i, 0)),
            pl.BlockSpec((D, D), lambda i: (0, 0)),
            pl.BlockSpec((D, D), lambda i: (0, 0)),
            pl.BlockSpec((1, D), lambda i: (0, 0)),
            pl.BlockSpec((1, D), lambda i: (0, 0)),
        ],
        out_specs=[
            pl.BlockSpec((bn, D), lambda i: (i, 0)),
            pl.BlockSpec((bn, D), lambda i: (i, 0)),
        ],
        out_shape=[
            jax.ShapeDtypeStruct((N, D), jnp.float32),
            jax.ShapeDtypeStruct((N, D), jnp.float32),
        ],
    )(nodes, wi2, wj2, bi2, bj2)


@functools.partial(
    pl.kernel,
    mesh=_sc_mesh,
    out_type=jax.ShapeDtypeStruct((E, D), jnp.float32),
    scratch_types=[
        pltpu.VMEM((CH,), jnp.int32),
        pltpu.VMEM((CH,), jnp.int32),
        pltpu.VMEM((CH, D), jnp.float32),
        pltpu.VMEM((CH, D), jnp.float32),
        pltpu.SemaphoreType.DMA,
    ],
)
def _edges_x(s_hbm, r_hbm, send_hbm, recv_hbm, x_hbm, idx_s, idx_r, s_buf, r_buf, sem):
    wid = lax.axis_index("s") * NC + lax.axis_index("c")

    @pl.loop(0, ITERS)
    def _(i):
        c = wid + i * NW

        @pl.when(c < NCHUNK)
        def _():
            base = c * CH
            pltpu.sync_copy(send_hbm.at[pl.ds(base, CH)], idx_s)
            pltpu.sync_copy(recv_hbm.at[pl.ds(base, CH)], idx_r)
            cp1 = pltpu.async_copy(s_hbm.at[idx_s], s_buf, sem)
            cp2 = pltpu.async_copy(r_hbm.at[idx_r], r_buf, sem)
            cp1.wait()
            cp2.wait()

            @pl.loop(0, CH)
            def _(e):
                for g in range(D // 16):
                    slc = (pl.ds(e, 1), pl.ds(g * 16, 16))
                    s_buf.at[*slc][...] = s_buf.at[*slc][...] + r_buf.at[*slc][...]

            pltpu.sync_copy(s_buf, x_hbm.at[pl.ds(base, CH)])


def _logits_body(x_ref, m_ref, z_ref):
    x = x_ref[...]
    m = x * jnp.tanh(jax.nn.softplus(x))
    l16 = jnp.dot(m, m_ref[...], preferred_element_type=jnp.float32)
    mask = (lax.broadcasted_iota(jnp.int32, l16.shape, 1) < H).astype(jnp.float32)
    z_ref[...] = jnp.exp(l16) * mask


def _logits(x, m16):
    be = 2560
    return pl.pallas_call(
        _logits_body,
        grid=(E // be,),
        in_specs=[
            pl.BlockSpec((be, D), lambda i: (i, 0)),
            pl.BlockSpec((D, 16), lambda i: (0, 0)),
        ],
        out_specs=pl.BlockSpec((be, 16), lambda i: (i, 0)),
        out_shape=jax.ShapeDtypeStruct((E, 16), jnp.float32),
    )(x, m16)


@functools.partial(
    pl.kernel,
    mesh=_sc_mesh,
    out_type=jax.ShapeDtypeStruct((NC, N, D), jnp.float32),
    scratch_types=[
        pltpu.VMEM((CH,), jnp.int32),
        pltpu.VMEM((CH, 16), jnp.float32),
        pltpu.VMEM((CH, D), jnp.float32),
        pltpu.VMEM_SHARED((N, D), jnp.float32),
    ],
)
def _denoms(recv_hbm, z_hbm, part_hbm, idx_r, z_buf, msg_buf, acc):
    cid = lax.axis_index("c")
    sid = lax.axis_index("s")
    wid = sid * NC + cid

    _zero_buf(msg_buf, CH, D)
    _zero_acc(sid, msg_buf, acc)
    plsc.subcore_barrier()

    @pl.loop(0, ITERS)
    def _(i):
        c = wid + i * NW

        @pl.when(c < NCHUNK)
        def _():
            base = c * CH
            pltpu.sync_copy(recv_hbm.at[pl.ds(base, CH)], idx_r)
            pltpu.sync_copy(z_hbm.at[pl.ds(base, CH)], z_buf)

            @pl.loop(0, CH)
            def _(e):
                msg_buf.at[pl.ds(e, 1), pl.ds(0, 16)][...] = z_buf.at[
                    pl.ds(e, 1), pl.ds(0, 16)
                ][...]

            pltpu.sync_copy(msg_buf, acc.at[idx_r], add=True)

    plsc.subcore_barrier()
    _dump_acc(cid, sid, acc, part_hbm)


@functools.partial(
    pl.kernel,
    mesh=_sc_mesh,
    out_type=jax.ShapeDtypeStruct((NC, N, D), jnp.float32),
    scratch_types=[
        pltpu.VMEM((CH,), jnp.int32),
        pltpu.VMEM((CH,), jnp.int32),
        pltpu.VMEM((CH, D), jnp.float32),
        pltpu.VMEM((CH, 16), jnp.float32),
        pltpu.VMEM((CH, D), jnp.float32),
        pltpu.VMEM_SHARED((N, D), jnp.float32),
        pltpu.SemaphoreType.DMA,
    ],
)
def _numers(
    s_hbm, send_hbm, recv_hbm, z_hbm, part_hbm,
    idx_s, idx_r, s_buf, z_buf, msg_buf, acc, sem,
):
    cid = lax.axis_index("c")
    sid = lax.axis_index("s")
    wid = sid * NC + cid

    _zero_buf(msg_buf, CH, D)
    _zero_acc(sid, msg_buf, acc)
    plsc.subcore_barrier()

    @pl.loop(0, ITERS)
    def _(i):
        c = wid + i * NW

        @pl.when(c < NCHUNK)
        def _():
            base = c * CH
            pltpu.sync_copy(send_hbm.at[pl.ds(base, CH)], idx_s)
            pltpu.sync_copy(recv_hbm.at[pl.ds(base, CH)], idx_r)
            pltpu.sync_copy(z_hbm.at[pl.ds(base, CH)], z_buf)
            pltpu.async_copy(s_hbm.at[idx_s], s_buf, sem).wait()

            @pl.loop(0, CH)
            def _(e):
                zr = z_buf.at[pl.ds(e, 1), pl.ds(0, 16)][...]
                for h in range(H):
                    zv = jnp.broadcast_to(zr[0, h], (1, 16))
                    for g in range(2):
                        slc = (pl.ds(e, 1), pl.ds(h * HD + g * 16, 16))
                        msg_buf.at[*slc][...] = s_buf.at[*slc][...] * zv

            pltpu.sync_copy(msg_buf, acc.at[idx_r], add=True)

    plsc.subcore_barrier()
    _dump_acc(cid, sid, acc, part_hbm)


def _final_body(num_ref, den_ref, o_ref):
    nump = num_ref[...]
    num = nump[0] + nump[1]
    denp = den_ref[...]
    den4 = (denp[0] + denp[1])[:, :H]
    row = lax.broadcasted_iota(jnp.int32, (H, D), 0)
    col = lax.broadcasted_iota(jnp.int32, (H, D), 1)
    pat = (col // HD == row).astype(jnp.float32)
    dent = jnp.dot(den4, pat, preferred_element_type=jnp.float32)
    o_ref[...] = num / jnp.maximum(dent, 1e-37)


def _finalize(num_parts, den_parts):
    bn = 2000
    return pl.pallas_call(
        _final_body,
        grid=(N // bn,),
        in_specs=[
            pl.BlockSpec((NC, bn, D), lambda i: (0, i, 0)),
            pl.BlockSpec((NC, bn, D), lambda i: (0, i, 0)),
        ],
        out_specs=pl.BlockSpec((bn, D), lambda i: (i, 0)),
        out_shape=jax.ShapeDtypeStruct((N, D), jnp.float32),
    )(num_parts, den_parts)


def kernel(nodes, senders, receivers, Wi, bi, Wj, bj, Wa, ba):
    wi2 = Wi.reshape(D, H * HD)
    wj2 = Wj.reshape(D, H * HD)
    bi2 = bi.reshape(1, H * HD)
    bj2 = bj.reshape(1, H * HD)
    eye = jnp.eye(H, dtype=jnp.float32)
    m4 = (eye[:, None, :] * Wa[:, 0][None, :, None]).reshape(H * HD, H)
    m16 = jnp.pad(m4, ((0, 0), (0, 16 - H)))

    s, r = _proj(nodes, wi2, wj2, bi2, bj2)
    x = _edges_x(s, r, senders, receivers)
    z = _logits(x, m16)
    den_parts = _denoms(receivers, z)
    num_parts = _numers(s, senders, receivers, z)
    return _finalize(num_parts, den_parts)
